# baseline (speedup 1.0000x reference)
import functools

import jax
import jax.numpy as jnp
from jax import lax
from jax.experimental import pallas as pl
from jax.experimental.pallas import tpu as pltpu
from jax.experimental.pallas import tpu_sc as plsc

_NW = 32
_GR = 128
_PREC = lax.Precision.DEFAULT


@functools.lru_cache(maxsize=None)
def _sc_gather_call(V, D, Rc, gr):
    mesh = plsc.VectorSubcoreMesh(core_axis_name="c", subcore_axis_name="s")
    npw = -(-Rc // _NW)

    @functools.partial(
        pl.kernel,
        out_type=jax.ShapeDtypeStruct((Rc * gr, D), jnp.float32),
        mesh=mesh,
        scratch_types=[
            pltpu.VMEM((npw, 1, gr), jnp.int32),
            pltpu.VMEM((gr, D), jnp.float32),
            pltpu.VMEM((gr, D), jnp.float32),
            pltpu.SemaphoreType.DMA,
            pltpu.SemaphoreType.DMA,
        ],
    )
    def gk(tab_hbm, idx_hbm, out_hbm, idx_v, buf0, buf1, g0, g1):
        wid = lax.axis_index("s") * 2 + lax.axis_index("c")
        base = wid * npw
        nv = jnp.clip(Rc - base, 0, npw)
        pltpu.sync_copy(idx_hbm.at[wid], idx_v)

        @pl.when(nv > 0)
        def _():
            pltpu.async_copy(tab_hbm.at[idx_v.at[0, 0]], buf0, g0)

        def body(p, carry):
            i = 2 * p

            @pl.when(i + 1 < nv)
            def _():
                pltpu.async_copy(tab_hbm.at[idx_v.at[i + 1, 0]], buf1, g1)

            @pl.when(i < nv)
            def _():
                pltpu.make_async_copy(tab_hbm.at[idx_v.at[i, 0]],
                                      buf0, g0).wait()
                pltpu.sync_copy(buf0, out_hbm.at[pl.ds((base + i) * gr, gr)])

            @pl.when(i + 2 < nv)
            def _():
                pltpu.async_copy(tab_hbm.at[idx_v.at[i + 2, 0]], buf0, g0)

            @pl.when(i + 1 < nv)
            def _():
                pltpu.make_async_copy(tab_hbm.at[idx_v.at[i + 1, 0]],
                                      buf1, g1).wait()
                pltpu.sync_copy(buf1,
                                out_hbm.at[pl.ds((base + i + 1) * gr, gr)])

            return carry

        lax.fori_loop(0, (npw + 1) // 2, body, 0)

    return gk


def _sc_gather(table, idx):
    V, D = table.shape
    gr = min(_GR, 32768 // D)
    R = idx.shape[0]
    Rc = R // gr
    npw = -(-Rc // _NW)
    idxp = jnp.pad(idx, (0, _NW * npw * gr - R)).reshape(_NW, npw, 1, gr)
    return _sc_gather_call(V, D, Rc, gr)(table, idxp)


def _kmaj_idx(idx, n_table):
    B, N, K = idx.shape
    off = (jnp.arange(B, dtype=jnp.int32) * n_table)[:, None, None]
    return jnp.transpose(idx.astype(jnp.int32) + off, (2, 0, 1)).reshape(-1)


def _dot(a, b):
    return jnp.dot(a, b, precision=_PREC, preferred_element_type=jnp.float32)


def _padr(w, rows):
    return jnp.pad(w, ((0, rows - w.shape[0]), (0, 0)))


def _padc(w, cols):
    return jnp.pad(w, ((0, 0), (0, cols - w.shape[1])))


def _full(shape):
    return pl.BlockSpec(shape, lambda b, n: (0,) * len(shape))


def _row_spec(P, C, nb):
    return pl.BlockSpec((P, C), lambda b, n: (b * nb + n, 0))


def _gac_pre(x, v, Wgs, bgs, Wap, Wah, P, NB):
    pooled = x.ndim == 3
    BN = x.shape[1] if pooled else x.shape[0]
    C = Wah.shape[1]
    nw = len(Wgs)
    grid = (BN // (P * NB), NB)

    def body(*refs):
        it = iter(refs)
        x_ref, v_ref = next(it), next(it)
        wg = [next(it) for _ in range(nw)]
        bg = [next(it) for _ in range(nw)]
        wap, wah = next(it), next(it)
        t_ref = next(it)
        h = jnp.max(x_ref[...], axis=0) if pooled else x_ref[...]
        for W, b in zip(wg, bg):
            h = jnp.maximum(_dot(h, W[...]) + b[...], 0.0)
        q = _dot(v_ref[...], wap[...]) + _dot(h, wah[...])
        t_ref[...] = jnp.concatenate([h, q], axis=-1)

    if pooled:
        x_spec = pl.BlockSpec((x.shape[0], P, x.shape[2]),
                              lambda b, n: (0, b * NB + n, 0))
    else:
        x_spec = _row_spec(P, x.shape[1], NB)
    in_specs = [x_spec, _row_spec(P, 3, NB)]
    in_specs += [_full(W.shape) for W in Wgs]
    in_specs += [_full(b.shape) for b in bgs]
    in_specs += [_full(Wap.shape), _full(Wah.shape)]
    return pl.pallas_call(
        body, grid=grid,
        in_specs=in_specs,
        out_specs=_row_spec(P, 2 * C, NB),
        out_shape=jax.ShapeDtypeStruct((BN, 2 * C), jnp.float32),
    )(x, v, *Wgs, *bgs, Wap, Wah)


def _gac_attn(G, T, Wo, bo, ba, P, NB):
    K, BN, C2 = G.shape
    C = C2 // 2
    Cout = Wo.shape[1]
    grid = (BN // (P * NB), NB)
    aligned = C % 128 == 0

    def body(g_ref, t_ref, wo_ref, bo_ref, ba_ref, out_ref):
        g = g_ref[...]
        if aligned:
            hj, qj = g[..., :C], g[..., C:]
            e = qj - t_ref[...][None, :, C:] + ba_ref[...][None]
        else:
            hj = g
            e = g - t_ref[...][None, :, :] + ba_ref[...][None]
        e = jnp.where(e >= 0, e, 0.2 * e)
        m = jnp.max(e, axis=0, keepdims=True)
        a = jnp.exp(e - m)
        an = a / jnp.sum(a, axis=0, keepdims=True)
        if not aligned:
            an = pltpu.roll(an, C, 2)
        agg = jnp.sum(an * hj, axis=0)
        out_ref[...] = jnp.maximum(_dot(agg, wo_ref[...]) + bo_ref[...], 0.0)

    in_specs = [pl.BlockSpec((K, P, C2), lambda b, n: (0, b * NB + n, 0)),
                _row_spec(P, C2, NB),
                _full(Wo.shape), _full(bo.shape), _full(ba.shape)]
    return pl.pallas_call(
        body, grid=grid, in_specs=in_specs,
        out_specs=_row_spec(P, Cout, NB),
        out_shape=jax.ShapeDtypeStruct((BN, Cout), jnp.float32),
    )(G, T, Wo, bo, ba)


def _upsample(vf, vcT, ff, fc, W0a, W0b, b0, W1, b1, P):
    BNf = vf.shape[0]
    C1 = ff.shape[1]
    Nc, C2 = fc.shape[1], fc.shape[2]
    H2 = W1.shape[1]
    B = fc.shape[0]
    NB = BNf // (B * P)
    grid = (B, NB)

    def body(vf_ref, vcT_ref, ff_ref, fc_ref, w0a, w0b, b0r, w1, b1r,
             out_ref):
        vfb = vf_ref[...]
        vct = vcT_ref[...]
        d = jnp.zeros((P, Nc), jnp.float32)
        for mdim in range(3):
            diff = vfb[:, mdim:mdim + 1] - vct[mdim:mdim + 1, :]
            d = d + diff * diff
        iota = lax.broadcasted_iota(jnp.int32, (P, Nc), 1)
        sels, ws = [], []
        dcur = d
        for _ in range(3):
            mval = jnp.min(dcur, axis=1, keepdims=True)
            idx = jnp.min(jnp.where(dcur == mval, iota, Nc), axis=1,
                          keepdims=True)
            sel = iota == idx
            sels.append(sel)
            ws.append(1.0 / (mval + 1e-8))
            dcur = jnp.where(sel, jnp.inf, dcur)
        tot = ws[0] + ws[1] + ws[2]
        wmat = jnp.zeros((P, Nc), jnp.float32)
        for sel, w in zip(sels, ws):
            wmat = wmat + jnp.where(sel, w / tot, 0.0)
        interp = _dot(wmat, fc_ref[...])
        xx = jnp.maximum(_dot(ff_ref[...], w0a[...]) +
                         _dot(interp, w0b[...]) + b0r[...], 0.0)
        out_ref[...] = jnp.maximum(_dot(xx, w1[...]) + b1r[...], 0.0)

    in_specs = [_row_spec(P, 3, NB),
                pl.BlockSpec((None, 3, Nc), lambda b, n: (b, 0, 0)),
                _row_spec(P, C1, NB),
                pl.BlockSpec((None, Nc, C2), lambda b, n: (b, 0, 0)),
                _full(W0a.shape), _full(W0b.shape), _full(b0.shape),
                _full(W1.shape), _full(b1.shape)]
    return pl.pallas_call(
        body, grid=grid, in_specs=in_specs,
        out_specs=_row_spec(P, H2, NB),
        out_shape=jax.ShapeDtypeStruct((BNf, H2), jnp.float32),
    )(vf, vcT, ff, fc, W0a, W0b, b0, W1, b1)


def _up_top3(vf, vcT, P):
    BNf = vf.shape[0]
    B, _, Nc = vcT.shape
    NB = BNf // (B * P)
    grid = (B, NB)

    def body(vf_ref, vcT_ref, idx_ref, w_ref):
        vfb = vf_ref[...]
        vct = vcT_ref[...]
        d = jnp.zeros((P, Nc), jnp.float32)
        for mdim in range(3):
            diff = vfb[:, mdim:mdim + 1] - vct[mdim:mdim + 1, :]
            d = d + diff * diff
        iota = lax.broadcasted_iota(jnp.int32, (P, Nc), 1)
        idxs, ws = [], []
        dcur = d
        for _ in range(3):
            mval = jnp.min(dcur, axis=1, keepdims=True)
            idx = jnp.min(jnp.where(dcur == mval, iota, Nc), axis=1,
                          keepdims=True)
            idxs.append(idx)
            ws.append(1.0 / (mval + 1e-8))
            dcur = jnp.where(iota == idx, jnp.inf, dcur)
        tot = ws[0] + ws[1] + ws[2]
        idx_ref[...] = jnp.concatenate(idxs, axis=1)
        w_ref[...] = jnp.concatenate([w / tot for w in ws], axis=1)

    return pl.pallas_call(
        body, grid=grid,
        in_specs=[_row_spec(P, 3, NB),
                  pl.BlockSpec((None, 3, Nc), lambda b, n: (b, 0, 0))],
        out_specs=[_row_spec(P, 3, NB), _row_spec(P, 3, NB)],
        out_shape=[jax.ShapeDtypeStruct((BNf, 3), jnp.int32),
                   jax.ShapeDtypeStruct((BNf, 3), jnp.float32)],
    )(vf, vcT)


def _up_interp(G3, w, ff, W0a, W0b, b0, W1, b1, P, NB):
    _, BNf, C2 = G3.shape
    C1 = ff.shape[1]
    H2 = W1.shape[1]
    grid = (BNf // (P * NB), NB)

    def body(g3_ref, w_ref, ff_ref, w0a, w0b, b0r, w1, b1r, out_ref):
        g3 = g3_ref[...]
        wv = w_ref[...]
        interp = (g3[0] * wv[:, 0:1] + g3[1] * wv[:, 1:2]
                  + g3[2] * wv[:, 2:3])
        xx = jnp.maximum(_dot(ff_ref[...], w0a[...]) +
                         _dot(interp, w0b[...]) + b0r[...], 0.0)
        out_ref[...] = jnp.maximum(_dot(xx, w1[...]) + b1r[...], 0.0)

    in_specs = [pl.BlockSpec((3, P, C2), lambda b, n: (0, b * NB + n, 0)),
                _row_spec(P, 3, NB), _row_spec(P, C1, NB),
                _full(W0a.shape), _full(W0b.shape), _full(b0.shape),
                _full(W1.shape), _full(b1.shape)]
    return pl.pallas_call(
        body, grid=grid, in_specs=in_specs,
        out_specs=_row_spec(P, H2, NB),
        out_shape=jax.ShapeDtypeStruct((BNf, H2), jnp.float32),
    )(G3, w, ff, W0a, W0b, b0, W1, b1)


def _head(f, inif, W1, b1, gamma, beta, W2p, Ssel, b2p, P, NB):
    BN, C = f.shape
    grid = (BN // (P * NB), NB)

    def body(f_ref, i_ref, w1, b1r, g, bt, w2, ssel, b2r, out_ref):
        x = _dot(f_ref[...], w1[...]) + b1r[...]
        x = jnp.maximum(g[...] * x + bt[...], 0.0)
        out_ref[...] = (_dot(x, w2[...]) + _dot(i_ref[...], ssel[...])
                        + b2r[...])

    in_specs = [_row_spec(P, C, NB), _row_spec(P, 6, NB),
                _full(W1.shape), _full(b1.shape), _full(gamma.shape),
                _full(beta.shape), _full(W2p.shape), _full(Ssel.shape),
                _full(b2p.shape)]
    return pl.pallas_call(
        body, grid=grid, in_specs=in_specs,
        out_specs=_row_spec(P, 128, NB),
        out_shape=jax.ShapeDtypeStruct((BN, 128), jnp.float32),
    )(f, inif, W1, b1, gamma, beta, W2p, Ssel, b2p)


def _final(G2, T2, Wr128, NC, P, NB):
    K, BN, _ = G2.shape
    grid = (BN // (P * NB), NB)

    def body(g2_ref, t2_ref, wr_ref, out_ref):
        g2 = g2_ref[...]
        dij = g2 - t2_ref[...][None, :, :]
        logits = _dot(dij.reshape(K * P, 128),
                      wr_ref[...]).reshape(K, P, 128)
        e = jnp.where(logits >= 0, logits, 0.2 * logits)
        m = jnp.max(e, axis=0, keepdims=True)
        a = jnp.exp(e - m)
        z = jnp.sum(a, axis=0)
        s = jnp.sum(a * g2, axis=0) / z
        mask = lax.broadcasted_iota(jnp.int32, (P, 128), 1) < NC
        zz = jnp.where(mask, s, -jnp.inf)
        mm = jnp.max(zz, axis=1, keepdims=True)
        lse = mm + jnp.log(jnp.sum(jnp.exp(zz - mm), axis=1, keepdims=True))
        out_ref[...] = s - lse

    in_specs = [pl.BlockSpec((K, P, 128), lambda b, n: (0, b * NB + n, 0)),
                _row_spec(P, 128, NB), _full(Wr128.shape)]
    return pl.pallas_call(
        body, grid=grid, in_specs=in_specs,
        out_specs=_row_spec(P, 128, NB),
        out_shape=jax.ShapeDtypeStruct((BN, 128), jnp.float32),
    )(G2, T2, Wr128)


_P_PRE = [1024, 512, 512, 128, 64]
_P_ATTN = [512, 128, 128, 32, 64]
_P_UP = [256, 256, 128, 128]


def kernel(features, vertex0, vertex1, vertex2, vertex3, vertex4,
           adjids0, adjids1, adjids2, adjids3, adjids4,
           cmap0, cmap1, cmap2, cmap3, params):
    vs = [vertex0, vertex1, vertex2, vertex3, vertex4]
    adjs = [adjids0, adjids1, adjids2, adjids3, adjids4]
    cmaps = [cmap0, cmap1, cmap2, cmap3]
    B = features.shape[0]
    ns = [v.shape[1] for v in vs]
    vflat = [v.reshape(B * v.shape[1], 3) for v in vs]

    inif = features[:, :, 0:6].reshape(B * ns[0], 6)
    x = features[:, :, 2:6].reshape(B * ns[0], 4)
    prd = []
    fo = None
    for l in range(5):
        gp = params['gac%d' % l]
        C = gp['Wa'].shape[1]
        aligned = C % 128 == 0
        Wgs = list(gp['Wg'])
        bgs = [b.reshape(1, -1) for b in gp['bg']]
        if Wgs[0].shape[0] != x.shape[-1]:
            Wgs[0] = _padr(Wgs[0], x.shape[-1])
        Wap, Wah = gp['Wa'][:3], gp['Wa'][3:]
        Cout = gp['Wo'].shape[1]
        Cot = max(Cout, 128)
        if aligned:
            ba = gp['ba'].reshape(1, -1)
            Wo = gp['Wo']
        else:
            ba = jnp.pad(gp['ba'], (C, 0)).reshape(1, -1)
            Wo = _padr(gp['Wo'], 2 * C)
        Wo = _padc(Wo, Cot)
        bo = _padc(gp['bo'].reshape(1, -1), Cot)
        nbl = ns[l] // _P_PRE[l]
        T = _gac_pre(x, vflat[l], Wgs, bgs, Wap, Wah, _P_PRE[l], nbl)
        K = adjs[l].shape[2]
        G = _sc_gather(T, _kmaj_idx(adjs[l], ns[l]))
        fo = _gac_attn(G.reshape(K, B * ns[l], 2 * C), T, Wo, bo, ba,
                       _P_ATTN[l], ns[l] // _P_ATTN[l])
        if l < 4:
            prd.append(fo)
            S = cmaps[l].shape[2]
            Gp = _sc_gather(fo, _kmaj_idx(cmaps[l], ns[l]))
            x = Gp.reshape(S, B * ns[l + 1], Cot)

    fcur = fo
    for l in [3, 2, 1, 0]:
        up = params['up%d' % l]
        C2 = fcur.shape[1]
        C1 = up['W'][0].shape[0] - C2
        W0a, W0b = up['W'][0][:C1], up['W'][0][C1:]
        if W0a.shape[0] != prd[l].shape[1]:
            W0a = _padr(W0a, prd[l].shape[1])
        if ns[l + 1] >= 1024:
            idx3, w3 = _up_top3(vflat[l], jnp.swapaxes(vs[l + 1], 1, 2),
                                _P_UP[l])
            gidx = jnp.transpose(
                idx3.reshape(B, ns[l], 3)
                + (jnp.arange(B, dtype=jnp.int32) * ns[l + 1])[:, None, None],
                (2, 0, 1)).reshape(-1)
            G3 = _sc_gather(fcur, gidx)
            fcur = _up_interp(G3.reshape(3, B * ns[l], C2), w3, prd[l],
                              W0a, W0b, up['b'][0].reshape(1, -1),
                              up['W'][1], up['b'][1].reshape(1, -1),
                              _P_UP[l], ns[l] // _P_UP[l])
        else:
            fcur = _upsample(vflat[l],
                             jnp.swapaxes(vs[l + 1], 1, 2), prd[l],
                             fcur.reshape(B, ns[l + 1], C2),
                             W0a, W0b, up['b'][0].reshape(1, -1),
                             up['W'][1], up['b'][1].reshape(1, -1),
                             _P_UP[l])

    NC = params['W2'].shape[1]
    W2p = _padc(params['W2'], 128)
    b2p = _padc(params['b2'].reshape(1, -1), 128)
    Ssel = jnp.pad(jnp.eye(6, dtype=jnp.float32), ((0, 0), (16, 106)))
    Wr128 = jnp.pad(params['Wr'], ((16, 106), (0, 128 - NC)))
    T2 = _head(fcur, inif, params['W1'], params['b1'].reshape(1, -1),
               params['gamma'].reshape(1, -1), params['beta'].reshape(1, -1),
               W2p, Ssel, b2p, 1024, ns[0] // 1024)
    K0 = adjs[0].shape[2]
    G2 = _sc_gather(T2, _kmaj_idx(adjs[0], ns[0]))
    out = _final(G2.reshape(K0, B * ns[0], 128), T2, Wr128, NC,
                 512, ns[0] // 512)
    return out.reshape(B, ns[0], 128)[:, :, :NC]


# speedup vs baseline: 1.0785x; 1.0785x over previous
import functools

import jax
import jax.numpy as jnp
from jax import lax
from jax.experimental import pallas as pl
from jax.experimental.pallas import tpu as pltpu
from jax.experimental.pallas import tpu_sc as plsc

_NW = 32
_GR = 128
_PREC = lax.Precision.DEFAULT


@functools.lru_cache(maxsize=None)
def _sc_gather_call(V, D, Rc, gr):
    mesh = plsc.VectorSubcoreMesh(core_axis_name="c", subcore_axis_name="s")
    npw = -(-Rc // _NW)

    @functools.partial(
        pl.kernel,
        out_type=jax.ShapeDtypeStruct((Rc * gr, D), jnp.float32),
        mesh=mesh,
        scratch_types=[
            pltpu.VMEM((npw, 1, gr), jnp.int32),
            pltpu.VMEM((gr, D), jnp.float32),
            pltpu.VMEM((gr, D), jnp.float32),
            pltpu.SemaphoreType.DMA,
            pltpu.SemaphoreType.DMA,
        ],
    )
    def gk(tab_hbm, idx_hbm, out_hbm, idx_v, buf0, buf1, g0, g1):
        wid = lax.axis_index("s") * 2 + lax.axis_index("c")
        base = wid * npw
        nv = jnp.clip(Rc - base, 0, npw)
        pltpu.sync_copy(idx_hbm.at[wid], idx_v)

        @pl.when(nv > 0)
        def _():
            pltpu.async_copy(tab_hbm.at[idx_v.at[0, 0]], buf0, g0)

        def body(p, carry):
            i = 2 * p

            @pl.when(i + 1 < nv)
            def _():
                pltpu.async_copy(tab_hbm.at[idx_v.at[i + 1, 0]], buf1, g1)

            @pl.when(i < nv)
            def _():
                pltpu.make_async_copy(tab_hbm.at[idx_v.at[i, 0]],
                                      buf0, g0).wait()
                pltpu.sync_copy(buf0, out_hbm.at[pl.ds((base + i) * gr, gr)])

            @pl.when(i + 2 < nv)
            def _():
                pltpu.async_copy(tab_hbm.at[idx_v.at[i + 2, 0]], buf0, g0)

            @pl.when(i + 1 < nv)
            def _():
                pltpu.make_async_copy(tab_hbm.at[idx_v.at[i + 1, 0]],
                                      buf1, g1).wait()
                pltpu.sync_copy(buf1,
                                out_hbm.at[pl.ds((base + i + 1) * gr, gr)])

            return carry

        lax.fori_loop(0, (npw + 1) // 2, body, 0)

    return gk


def _sc_gather(table, idx):
    V, D = table.shape
    gr = min(_GR, 32768 // D)
    R = idx.shape[0]
    Rc = R // gr
    npw = -(-Rc // _NW)
    idxp = jnp.pad(idx, (0, _NW * npw * gr - R)).reshape(_NW, npw, 1, gr)
    return _sc_gather_call(V, D, Rc, gr)(table, idxp)


def _kmaj_idx(idx, n_table):
    B, N, K = idx.shape
    off = (jnp.arange(B, dtype=jnp.int32) * n_table)[:, None, None]
    return jnp.transpose(idx.astype(jnp.int32) + off, (2, 0, 1)).reshape(-1)


def _dot(a, b):
    return jnp.dot(a, b, precision=_PREC, preferred_element_type=jnp.float32)


def _padr(w, rows):
    return jnp.pad(w, ((0, rows - w.shape[0]), (0, 0)))


def _padc(w, cols):
    return jnp.pad(w, ((0, 0), (0, cols - w.shape[1])))


def _full(shape):
    return pl.BlockSpec(shape, lambda b, n: (0,) * len(shape))


def _row_spec(P, C, nb):
    return pl.BlockSpec((P, C), lambda b, n: (b * nb + n, 0))


def _gac_pre(x, v, Wgs, bgs, Wap, Wah, P, NB):
    pooled = x.ndim == 3
    BN = x.shape[1] if pooled else x.shape[0]
    C = Wah.shape[1]
    nw = len(Wgs)
    grid = (BN // (P * NB), NB)

    def body(*refs):
        it = iter(refs)
        x_ref, v_ref = next(it), next(it)
        wg = [next(it) for _ in range(nw)]
        bg = [next(it) for _ in range(nw)]
        wap, wah = next(it), next(it)
        t_ref = next(it)
        h = jnp.max(x_ref[...], axis=0) if pooled else x_ref[...]
        for W, b in zip(wg, bg):
            h = jnp.maximum(_dot(h, W[...]) + b[...], 0.0)
        q = _dot(v_ref[...], wap[...]) + _dot(h, wah[...])
        t_ref[...] = jnp.concatenate([h, q], axis=-1)

    if pooled:
        x_spec = pl.BlockSpec((x.shape[0], P, x.shape[2]),
                              lambda b, n: (0, b * NB + n, 0))
    else:
        x_spec = _row_spec(P, x.shape[1], NB)
    in_specs = [x_spec, _row_spec(P, 3, NB)]
    in_specs += [_full(W.shape) for W in Wgs]
    in_specs += [_full(b.shape) for b in bgs]
    in_specs += [_full(Wap.shape), _full(Wah.shape)]
    return pl.pallas_call(
        body, grid=grid,
        in_specs=in_specs,
        out_specs=_row_spec(P, 2 * C, NB),
        out_shape=jax.ShapeDtypeStruct((BN, 2 * C), jnp.float32),
    )(x, v, *Wgs, *bgs, Wap, Wah)


def _gac_attn(G, T, Wo, bo, ba, P, NB):
    K, BN, C2 = G.shape
    C = C2 // 2
    Cout = Wo.shape[1]
    grid = (BN // (P * NB), NB)
    aligned = C % 128 == 0

    def body(g_ref, t_ref, wo_ref, bo_ref, ba_ref, out_ref):
        g = g_ref[...]
        if aligned:
            hj, qj = g[..., :C], g[..., C:]
            e = qj - t_ref[...][None, :, C:] + ba_ref[...][None]
        else:
            hj = g
            e = g - t_ref[...][None, :, :] + ba_ref[...][None]
        e = jnp.where(e >= 0, e, 0.2 * e)
        m = jnp.max(e, axis=0, keepdims=True)
        a = jnp.exp(e - m)
        an = a / jnp.sum(a, axis=0, keepdims=True)
        if not aligned:
            an = pltpu.roll(an, C, 2)
        agg = jnp.sum(an * hj, axis=0)
        out_ref[...] = jnp.maximum(_dot(agg, wo_ref[...]) + bo_ref[...], 0.0)

    in_specs = [pl.BlockSpec((K, P, C2), lambda b, n: (0, b * NB + n, 0)),
                _row_spec(P, C2, NB),
                _full(Wo.shape), _full(bo.shape), _full(ba.shape)]
    return pl.pallas_call(
        body, grid=grid, in_specs=in_specs,
        out_specs=_row_spec(P, Cout, NB),
        out_shape=jax.ShapeDtypeStruct((BN, Cout), jnp.float32),
    )(G, T, Wo, bo, ba)


def _upsample(vf, vcT, ff, fc, W0a, W0b, b0, W1, b1, P):
    BNf = vf.shape[0]
    C1 = ff.shape[1]
    Nc, C2 = fc.shape[1], fc.shape[2]
    H2 = W1.shape[1]
    B = fc.shape[0]
    NB = BNf // (B * P)
    grid = (B, NB)

    def body(vf_ref, vcT_ref, ff_ref, fc_ref, w0a, w0b, b0r, w1, b1r,
             out_ref):
        vfb = vf_ref[...]
        vct = vcT_ref[...]
        d = jnp.zeros((P, Nc), jnp.float32)
        for mdim in range(3):
            diff = vfb[:, mdim:mdim + 1] - vct[mdim:mdim + 1, :]
            d = d + diff * diff
        iota = lax.broadcasted_iota(jnp.int32, (P, Nc), 1)
        sels, ws = [], []
        dcur = d
        for _ in range(3):
            mval = jnp.min(dcur, axis=1, keepdims=True)
            idx = jnp.min(jnp.where(dcur == mval, iota, Nc), axis=1,
                          keepdims=True)
            sel = iota == idx
            sels.append(sel)
            ws.append(1.0 / (mval + 1e-8))
            dcur = jnp.where(sel, jnp.inf, dcur)
        tot = ws[0] + ws[1] + ws[2]
        wmat = jnp.zeros((P, Nc), jnp.float32)
        for sel, w in zip(sels, ws):
            wmat = wmat + jnp.where(sel, w / tot, 0.0)
        interp = _dot(wmat, fc_ref[...])
        xx = jnp.maximum(_dot(ff_ref[...], w0a[...]) +
                         _dot(interp, w0b[...]) + b0r[...], 0.0)
        out_ref[...] = jnp.maximum(_dot(xx, w1[...]) + b1r[...], 0.0)

    in_specs = [_row_spec(P, 3, NB),
                pl.BlockSpec((None, 3, Nc), lambda b, n: (b, 0, 0)),
                _row_spec(P, C1, NB),
                pl.BlockSpec((None, Nc, C2), lambda b, n: (b, 0, 0)),
                _full(W0a.shape), _full(W0b.shape), _full(b0.shape),
                _full(W1.shape), _full(b1.shape)]
    return pl.pallas_call(
        body, grid=grid, in_specs=in_specs,
        out_specs=_row_spec(P, H2, NB),
        out_shape=jax.ShapeDtypeStruct((BNf, H2), jnp.float32),
    )(vf, vcT, ff, fc, W0a, W0b, b0, W1, b1)


def _head(f, inif, W1, b1, gamma, beta, W2p, Ssel, b2p, P, NB):
    BN, C = f.shape
    grid = (BN // (P * NB), NB)

    def body(f_ref, i_ref, w1, b1r, g, bt, w2, ssel, b2r, out_ref):
        x = _dot(f_ref[...], w1[...]) + b1r[...]
        x = jnp.maximum(g[...] * x + bt[...], 0.0)
        out_ref[...] = (_dot(x, w2[...]) + _dot(i_ref[...], ssel[...])
                        + b2r[...])

    in_specs = [_row_spec(P, C, NB), _row_spec(P, 6, NB),
                _full(W1.shape), _full(b1.shape), _full(gamma.shape),
                _full(beta.shape), _full(W2p.shape), _full(Ssel.shape),
                _full(b2p.shape)]
    return pl.pallas_call(
        body, grid=grid, in_specs=in_specs,
        out_specs=_row_spec(P, 128, NB),
        out_shape=jax.ShapeDtypeStruct((BN, 128), jnp.float32),
    )(f, inif, W1, b1, gamma, beta, W2p, Ssel, b2p)


def _final(G2, T2, Wr128, NC, P, NB):
    K, BN, _ = G2.shape
    grid = (BN // (P * NB), NB)

    def body(g2_ref, t2_ref, wr_ref, out_ref):
        g2 = g2_ref[...]
        dij = g2 - t2_ref[...][None, :, :]
        logits = _dot(dij.reshape(K * P, 128),
                      wr_ref[...]).reshape(K, P, 128)
        e = jnp.where(logits >= 0, logits, 0.2 * logits)
        m = jnp.max(e, axis=0, keepdims=True)
        a = jnp.exp(e - m)
        z = jnp.sum(a, axis=0)
        s = jnp.sum(a * g2, axis=0) / z
        mask = lax.broadcasted_iota(jnp.int32, (P, 128), 1) < NC
        zz = jnp.where(mask, s, -jnp.inf)
        mm = jnp.max(zz, axis=1, keepdims=True)
        lse = mm + jnp.log(jnp.sum(jnp.exp(zz - mm), axis=1, keepdims=True))
        out_ref[...] = s - lse

    in_specs = [pl.BlockSpec((K, P, 128), lambda b, n: (0, b * NB + n, 0)),
                _row_spec(P, 128, NB), _full(Wr128.shape)]
    return pl.pallas_call(
        body, grid=grid, in_specs=in_specs,
        out_specs=_row_spec(P, 128, NB),
        out_shape=jax.ShapeDtypeStruct((BN, 128), jnp.float32),
    )(G2, T2, Wr128)


_P_PRE = [1024, 512, 512, 128, 64]
_P_ATTN = [512, 128, 128, 32, 64]
_P_UP = [256, 256, 128, 128]


def kernel(features, vertex0, vertex1, vertex2, vertex3, vertex4,
           adjids0, adjids1, adjids2, adjids3, adjids4,
           cmap0, cmap1, cmap2, cmap3, params):
    args = (features, vertex0, vertex1, vertex2, vertex3, vertex4,
            adjids0, adjids1, adjids2, adjids3, adjids4,
            cmap0, cmap1, cmap2, cmap3)
    outs = [_forward(*(a[b:b + 1] for a in args), params)
            for b in range(features.shape[0])]
    return jnp.concatenate(outs, axis=0)


def _forward(features, vertex0, vertex1, vertex2, vertex3, vertex4,
             adjids0, adjids1, adjids2, adjids3, adjids4,
             cmap0, cmap1, cmap2, cmap3, params):
    vs = [vertex0, vertex1, vertex2, vertex3, vertex4]
    adjs = [adjids0, adjids1, adjids2, adjids3, adjids4]
    cmaps = [cmap0, cmap1, cmap2, cmap3]
    B = features.shape[0]
    ns = [v.shape[1] for v in vs]
    vflat = [v.reshape(B * v.shape[1], 3) for v in vs]

    inif = features[:, :, 0:6].reshape(B * ns[0], 6)
    x = features[:, :, 2:6].reshape(B * ns[0], 4)
    prd = []
    fo = None
    for l in range(5):
        gp = params['gac%d' % l]
        C = gp['Wa'].shape[1]
        aligned = C % 128 == 0
        Wgs = list(gp['Wg'])
        bgs = [b.reshape(1, -1) for b in gp['bg']]
        if Wgs[0].shape[0] != x.shape[-1]:
            Wgs[0] = _padr(Wgs[0], x.shape[-1])
        Wap, Wah = gp['Wa'][:3], gp['Wa'][3:]
        Cout = gp['Wo'].shape[1]
        Cot = max(Cout, 128)
        if aligned:
            ba = gp['ba'].reshape(1, -1)
            Wo = gp['Wo']
        else:
            ba = jnp.pad(gp['ba'], (C, 0)).reshape(1, -1)
            Wo = _padr(gp['Wo'], 2 * C)
        Wo = _padc(Wo, Cot)
        bo = _padc(gp['bo'].reshape(1, -1), Cot)
        nbl = ns[l] // _P_PRE[l]
        T = _gac_pre(x, vflat[l], Wgs, bgs, Wap, Wah, _P_PRE[l], nbl)
        K = adjs[l].shape[2]
        G = _sc_gather(T, _kmaj_idx(adjs[l], ns[l]))
        fo = _gac_attn(G.reshape(K, B * ns[l], 2 * C), T, Wo, bo, ba,
                       _P_ATTN[l], ns[l] // _P_ATTN[l])
        if l < 4:
            prd.append(fo)
            S = cmaps[l].shape[2]
            Gp = _sc_gather(fo, _kmaj_idx(cmaps[l], ns[l]))
            x = Gp.reshape(S, B * ns[l + 1], Cot)

    fcur = fo
    for l in [3, 2, 1, 0]:
        up = params['up%d' % l]
        C2 = fcur.shape[1]
        C1 = up['W'][0].shape[0] - C2
        W0a, W0b = up['W'][0][:C1], up['W'][0][C1:]
        if W0a.shape[0] != prd[l].shape[1]:
            W0a = _padr(W0a, prd[l].shape[1])
        fcur = _upsample(vflat[l],
                         jnp.swapaxes(vs[l + 1], 1, 2), prd[l],
                         fcur.reshape(B, ns[l + 1], C2),
                         W0a, W0b, up['b'][0].reshape(1, -1),
                         up['W'][1], up['b'][1].reshape(1, -1), _P_UP[l])

    NC = params['W2'].shape[1]
    W2p = _padc(params['W2'], 128)
    b2p = _padc(params['b2'].reshape(1, -1), 128)
    Ssel = jnp.pad(jnp.eye(6, dtype=jnp.float32), ((0, 0), (16, 106)))
    Wr128 = jnp.pad(params['Wr'], ((16, 106), (0, 128 - NC)))
    T2 = _head(fcur, inif, params['W1'], params['b1'].reshape(1, -1),
               params['gamma'].reshape(1, -1), params['beta'].reshape(1, -1),
               W2p, Ssel, b2p, 1024, ns[0] // 1024)
    K0 = adjs[0].shape[2]
    G2 = _sc_gather(T2, _kmaj_idx(adjs[0], ns[0]))
    out = _final(G2.reshape(K0, B * ns[0], 128), T2, Wr128, NC,
                 512, ns[0] // 512)
    return out.reshape(B, ns[0], 128)[:, :, :NC]


# speedup vs baseline: 1.0844x; 1.0055x over previous
import functools

import jax
import jax.numpy as jnp
from jax import lax
from jax.experimental import pallas as pl
from jax.experimental.pallas import tpu as pltpu
from jax.experimental.pallas import tpu_sc as plsc

_NW = 32
_GR = 128
_PREC = lax.Precision.DEFAULT


@functools.lru_cache(maxsize=None)
def _sc_gather_call(V, D, Rc, gr):
    mesh = plsc.VectorSubcoreMesh(core_axis_name="c", subcore_axis_name="s")
    npw = -(-Rc // _NW)

    @functools.partial(
        pl.kernel,
        out_type=jax.ShapeDtypeStruct((Rc * gr, D), jnp.float32),
        mesh=mesh,
        scratch_types=[
            pltpu.VMEM((npw, 1, gr), jnp.int32),
            pltpu.VMEM((gr, D), jnp.float32),
            pltpu.VMEM((gr, D), jnp.float32),
            pltpu.SemaphoreType.DMA,
            pltpu.SemaphoreType.DMA,
        ],
    )
    def gk(tab_hbm, idx_hbm, out_hbm, idx_v, buf0, buf1, g0, g1):
        wid = lax.axis_index("s") * 2 + lax.axis_index("c")
        base = wid * npw
        nv = jnp.clip(Rc - base, 0, npw)
        pltpu.sync_copy(idx_hbm.at[wid], idx_v)

        @pl.when(nv > 0)
        def _():
            pltpu.async_copy(tab_hbm.at[idx_v.at[0, 0]], buf0, g0)

        def body(p, carry):
            i = 2 * p

            @pl.when(i + 1 < nv)
            def _():
                pltpu.async_copy(tab_hbm.at[idx_v.at[i + 1, 0]], buf1, g1)

            @pl.when(i < nv)
            def _():
                pltpu.make_async_copy(tab_hbm.at[idx_v.at[i, 0]],
                                      buf0, g0).wait()
                pltpu.sync_copy(buf0, out_hbm.at[pl.ds((base + i) * gr, gr)])

            @pl.when(i + 2 < nv)
            def _():
                pltpu.async_copy(tab_hbm.at[idx_v.at[i + 2, 0]], buf0, g0)

            @pl.when(i + 1 < nv)
            def _():
                pltpu.make_async_copy(tab_hbm.at[idx_v.at[i + 1, 0]],
                                      buf1, g1).wait()
                pltpu.sync_copy(buf1,
                                out_hbm.at[pl.ds((base + i + 1) * gr, gr)])

            return carry

        lax.fori_loop(0, (npw + 1) // 2, body, 0)

    return gk


def _sc_gather(table, idx):
    V, D = table.shape
    gr = min(_GR, 32768 // D)
    R = idx.shape[0]
    Rc = R // gr
    npw = -(-Rc // _NW)
    idxp = jnp.pad(idx, (0, _NW * npw * gr - R)).reshape(_NW, npw, 1, gr)
    return _sc_gather_call(V, D, Rc, gr)(table, idxp)


def _kmaj_idx(idx, n_table):
    B, N, K = idx.shape
    off = (jnp.arange(B, dtype=jnp.int32) * n_table)[:, None, None]
    return jnp.transpose(idx.astype(jnp.int32) + off, (2, 0, 1)).reshape(-1)


def _dot(a, b):
    return jnp.dot(a, b, precision=_PREC, preferred_element_type=jnp.float32)


def _padr(w, rows):
    return jnp.pad(w, ((0, rows - w.shape[0]), (0, 0)))


def _padc(w, cols):
    return jnp.pad(w, ((0, 0), (0, cols - w.shape[1])))


def _full(shape):
    return pl.BlockSpec(shape, lambda b, n: (0,) * len(shape))


def _row_spec(P, C, nb):
    return pl.BlockSpec((P, C), lambda b, n: (b * nb + n, 0))


def _gac_pre(x, v, Wgs, bgs, Wap, Wah, P, NB):
    pooled = x.ndim == 3
    BN = x.shape[1] if pooled else x.shape[0]
    C = Wah.shape[1]
    nw = len(Wgs)
    grid = (BN // (P * NB), NB)

    def body(*refs):
        it = iter(refs)
        x_ref, v_ref = next(it), next(it)
        wg = [next(it) for _ in range(nw)]
        bg = [next(it) for _ in range(nw)]
        wap, wah = next(it), next(it)
        t_ref = next(it)
        h = jnp.max(x_ref[...], axis=0) if pooled else x_ref[...]
        for W, b in zip(wg, bg):
            h = jnp.maximum(_dot(h, W[...]) + b[...], 0.0)
        q = _dot(v_ref[...], wap[...]) + _dot(h, wah[...])
        t_ref[...] = jnp.concatenate([h, q], axis=-1)

    if pooled:
        x_spec = pl.BlockSpec((x.shape[0], P, x.shape[2]),
                              lambda b, n: (0, b * NB + n, 0))
    else:
        x_spec = _row_spec(P, x.shape[1], NB)
    in_specs = [x_spec, _row_spec(P, 3, NB)]
    in_specs += [_full(W.shape) for W in Wgs]
    in_specs += [_full(b.shape) for b in bgs]
    in_specs += [_full(Wap.shape), _full(Wah.shape)]
    return pl.pallas_call(
        body, grid=grid,
        in_specs=in_specs,
        out_specs=_row_spec(P, 2 * C, NB),
        out_shape=jax.ShapeDtypeStruct((BN, 2 * C), jnp.float32),
    )(x, v, *Wgs, *bgs, Wap, Wah)


def _gac_attn(G, T, Wo, bo, ba, P, NB):
    K, BN, C2 = G.shape
    C = C2 // 2
    Cout = Wo.shape[1]
    grid = (BN // (P * NB), NB)
    aligned = C % 128 == 0

    def body(g_ref, t_ref, wo_ref, bo_ref, ba_ref, out_ref):
        g = g_ref[...]
        if aligned:
            hj, qj = g[..., :C], g[..., C:]
            e = qj - t_ref[...][None, :, C:] + ba_ref[...][None]
        else:
            hj = g
            e = g - t_ref[...][None, :, :] + ba_ref[...][None]
        e = jnp.where(e >= 0, e, 0.2 * e)
        m = jnp.max(e, axis=0, keepdims=True)
        a = jnp.exp(e - m)
        an = a / jnp.sum(a, axis=0, keepdims=True)
        if not aligned:
            an = pltpu.roll(an, C, 2)
        agg = jnp.sum(an * hj, axis=0)
        out_ref[...] = jnp.maximum(_dot(agg, wo_ref[...]) + bo_ref[...], 0.0)

    in_specs = [pl.BlockSpec((K, P, C2), lambda b, n: (0, b * NB + n, 0)),
                _row_spec(P, C2, NB),
                _full(Wo.shape), _full(bo.shape), _full(ba.shape)]
    return pl.pallas_call(
        body, grid=grid, in_specs=in_specs,
        out_specs=_row_spec(P, Cout, NB),
        out_shape=jax.ShapeDtypeStruct((BN, Cout), jnp.float32),
    )(G, T, Wo, bo, ba)


def _upsample(vf, vcT, ff, fc, W0a, W0b, b0, W1, b1, P):
    BNf = vf.shape[0]
    C1 = ff.shape[1]
    Nc, C2 = fc.shape[1], fc.shape[2]
    H2 = W1.shape[1]
    B = fc.shape[0]
    NB = BNf // (B * P)
    grid = (B, NB)

    def body(vf_ref, vcT_ref, ff_ref, fc_ref, w0a, w0b, b0r, w1, b1r,
             out_ref):
        vfb = vf_ref[...]
        vct = vcT_ref[...]
        d = jnp.zeros((P, Nc), jnp.float32)
        for mdim in range(3):
            diff = vfb[:, mdim:mdim + 1] - vct[mdim:mdim + 1, :]
            d = d + diff * diff
        iota = lax.broadcasted_iota(jnp.int32, (P, Nc), 1)
        sels, ws = [], []
        dcur = d
        for _ in range(3):
            mval = jnp.min(dcur, axis=1, keepdims=True)
            idx = jnp.min(jnp.where(dcur == mval, iota, Nc), axis=1,
                          keepdims=True)
            sel = iota == idx
            sels.append(sel)
            ws.append(1.0 / (mval + 1e-8))
            dcur = jnp.where(sel, jnp.inf, dcur)
        tot = ws[0] + ws[1] + ws[2]
        wmat = jnp.zeros((P, Nc), jnp.float32)
        for sel, w in zip(sels, ws):
            wmat = wmat + jnp.where(sel, w / tot, 0.0)
        interp = _dot(wmat, fc_ref[...])
        xx = jnp.maximum(_dot(ff_ref[...], w0a[...]) +
                         _dot(interp, w0b[...]) + b0r[...], 0.0)
        out_ref[...] = jnp.maximum(_dot(xx, w1[...]) + b1r[...], 0.0)

    in_specs = [_row_spec(P, 3, NB),
                pl.BlockSpec((None, 3, Nc), lambda b, n: (b, 0, 0)),
                _row_spec(P, C1, NB),
                pl.BlockSpec((None, Nc, C2), lambda b, n: (b, 0, 0)),
                _full(W0a.shape), _full(W0b.shape), _full(b0.shape),
                _full(W1.shape), _full(b1.shape)]
    return pl.pallas_call(
        body, grid=grid, in_specs=in_specs,
        out_specs=_row_spec(P, H2, NB),
        out_shape=jax.ShapeDtypeStruct((BNf, H2), jnp.float32),
    )(vf, vcT, ff, fc, W0a, W0b, b0, W1, b1)


def _head(f, inif, W1, b1, gamma, beta, W2p, Ssel, b2p, P, NB):
    BN, C = f.shape
    grid = (BN // (P * NB), NB)

    def body(f_ref, i_ref, w1, b1r, g, bt, w2, ssel, b2r, out_ref):
        x = _dot(f_ref[...], w1[...]) + b1r[...]
        x = jnp.maximum(g[...] * x + bt[...], 0.0)
        out_ref[...] = (_dot(x, w2[...]) + _dot(i_ref[...], ssel[...])
                        + b2r[...])

    in_specs = [_row_spec(P, C, NB), _row_spec(P, 6, NB),
                _full(W1.shape), _full(b1.shape), _full(gamma.shape),
                _full(beta.shape), _full(W2p.shape), _full(Ssel.shape),
                _full(b2p.shape)]
    return pl.pallas_call(
        body, grid=grid, in_specs=in_specs,
        out_specs=_row_spec(P, 128, NB),
        out_shape=jax.ShapeDtypeStruct((BN, 128), jnp.float32),
    )(f, inif, W1, b1, gamma, beta, W2p, Ssel, b2p)


def _final(G2, T2, Wr128, NC, P, NB):
    K, BN, _ = G2.shape
    grid = (BN // (P * NB), NB)

    def body(g2_ref, t2_ref, wr_ref, out_ref):
        g2 = g2_ref[...]
        dij = g2 - t2_ref[...][None, :, :]
        logits = _dot(dij.reshape(K * P, 128),
                      wr_ref[...]).reshape(K, P, 128)
        e = jnp.where(logits >= 0, logits, 0.2 * logits)
        m = jnp.max(e, axis=0, keepdims=True)
        a = jnp.exp(e - m)
        z = jnp.sum(a, axis=0)
        s = jnp.sum(a * g2, axis=0) / z
        mask = lax.broadcasted_iota(jnp.int32, (P, 128), 1) < NC
        zz = jnp.where(mask, s, -jnp.inf)
        mm = jnp.max(zz, axis=1, keepdims=True)
        lse = mm + jnp.log(jnp.sum(jnp.exp(zz - mm), axis=1, keepdims=True))
        out_ref[...] = s - lse

    in_specs = [pl.BlockSpec((K, P, 128), lambda b, n: (0, b * NB + n, 0)),
                _row_spec(P, 128, NB), _full(Wr128.shape)]
    return pl.pallas_call(
        body, grid=grid, in_specs=in_specs,
        out_specs=_row_spec(P, 128, NB),
        out_shape=jax.ShapeDtypeStruct((BN, 128), jnp.float32),
    )(G2, T2, Wr128)


_P_PRE = [1024, 512, 512, 128, 64]
_P_ATTN = [1024, 256, 128, 32, 64]
_P_UP = [256, 256, 128, 128]


def kernel(features, vertex0, vertex1, vertex2, vertex3, vertex4,
           adjids0, adjids1, adjids2, adjids3, adjids4,
           cmap0, cmap1, cmap2, cmap3, params):
    args = (features, vertex0, vertex1, vertex2, vertex3, vertex4,
            adjids0, adjids1, adjids2, adjids3, adjids4,
            cmap0, cmap1, cmap2, cmap3)
    outs = [_forward(*(a[b:b + 1] for a in args), params)
            for b in range(features.shape[0])]
    return jnp.concatenate(outs, axis=0)


def _forward(features, vertex0, vertex1, vertex2, vertex3, vertex4,
             adjids0, adjids1, adjids2, adjids3, adjids4,
             cmap0, cmap1, cmap2, cmap3, params):
    vs = [vertex0, vertex1, vertex2, vertex3, vertex4]
    adjs = [adjids0, adjids1, adjids2, adjids3, adjids4]
    cmaps = [cmap0, cmap1, cmap2, cmap3]
    B = features.shape[0]
    ns = [v.shape[1] for v in vs]
    vflat = [v.reshape(B * v.shape[1], 3) for v in vs]

    inif = features[:, :, 0:6].reshape(B * ns[0], 6)
    x = features[:, :, 2:6].reshape(B * ns[0], 4)
    prd = []
    fo = None
    for l in range(5):
        gp = params['gac%d' % l]
        C = gp['Wa'].shape[1]
        aligned = C % 128 == 0
        Wgs = list(gp['Wg'])
        bgs = [b.reshape(1, -1) for b in gp['bg']]
        if Wgs[0].shape[0] != x.shape[-1]:
            Wgs[0] = _padr(Wgs[0], x.shape[-1])
        Wap, Wah = gp['Wa'][:3], gp['Wa'][3:]
        Cout = gp['Wo'].shape[1]
        Cot = max(Cout, 128)
        if aligned:
            ba = gp['ba'].reshape(1, -1)
            Wo = gp['Wo']
        else:
            ba = jnp.pad(gp['ba'], (C, 0)).reshape(1, -1)
            Wo = _padr(gp['Wo'], 2 * C)
        Wo = _padc(Wo, Cot)
        bo = _padc(gp['bo'].reshape(1, -1), Cot)
        nbl = ns[l] // _P_PRE[l]
        T = _gac_pre(x, vflat[l], Wgs, bgs, Wap, Wah, _P_PRE[l], nbl)
        K = adjs[l].shape[2]
        G = _sc_gather(T, _kmaj_idx(adjs[l], ns[l]))
        fo = _gac_attn(G.reshape(K, B * ns[l], 2 * C), T, Wo, bo, ba,
                       _P_ATTN[l], ns[l] // _P_ATTN[l])
        if l < 4:
            prd.append(fo)
            S = cmaps[l].shape[2]
            Gp = _sc_gather(fo, _kmaj_idx(cmaps[l], ns[l]))
            x = Gp.reshape(S, B * ns[l + 1], Cot)

    fcur = fo
    for l in [3, 2, 1, 0]:
        up = params['up%d' % l]
        C2 = fcur.shape[1]
        C1 = up['W'][0].shape[0] - C2
        W0a, W0b = up['W'][0][:C1], up['W'][0][C1:]
        if W0a.shape[0] != prd[l].shape[1]:
            W0a = _padr(W0a, prd[l].shape[1])
        fcur = _upsample(vflat[l],
                         jnp.swapaxes(vs[l + 1], 1, 2), prd[l],
                         fcur.reshape(B, ns[l + 1], C2),
                         W0a, W0b, up['b'][0].reshape(1, -1),
                         up['W'][1], up['b'][1].reshape(1, -1), _P_UP[l])

    NC = params['W2'].shape[1]
    W2p = _padc(params['W2'], 128)
    b2p = _padc(params['b2'].reshape(1, -1), 128)
    Ssel = jnp.pad(jnp.eye(6, dtype=jnp.float32), ((0, 0), (16, 106)))
    Wr128 = jnp.pad(params['Wr'], ((16, 106), (0, 128 - NC)))
    T2 = _head(fcur, inif, params['W1'], params['b1'].reshape(1, -1),
               params['gamma'].reshape(1, -1), params['beta'].reshape(1, -1),
               W2p, Ssel, b2p, 1024, ns[0] // 1024)
    K0 = adjs[0].shape[2]
    G2 = _sc_gather(T2, _kmaj_idx(adjs[0], ns[0]))
    out = _final(G2.reshape(K0, B * ns[0], 128), T2, Wr128, NC,
                 512, ns[0] // 512)
    return out.reshape(B, ns[0], 128)[:, :, :NC]


# speedup vs baseline: 1.0999x; 1.0143x over previous
import functools

import jax
import jax.numpy as jnp
from jax import lax
from jax.experimental import pallas as pl
from jax.experimental.pallas import tpu as pltpu
from jax.experimental.pallas import tpu_sc as plsc

_NW = 32
_GR = 128
_PREC = lax.Precision.DEFAULT


@functools.lru_cache(maxsize=None)
def _sc_gather_call(V, D, Rc, gr):
    mesh = plsc.VectorSubcoreMesh(core_axis_name="c", subcore_axis_name="s")
    npw = -(-Rc // _NW)

    @functools.partial(
        pl.kernel,
        out_type=jax.ShapeDtypeStruct((Rc * gr, D), jnp.float32),
        mesh=mesh,
        scratch_types=[
            pltpu.VMEM((npw, 1, gr), jnp.int32),
            pltpu.VMEM((gr, D), jnp.float32),
            pltpu.VMEM((gr, D), jnp.float32),
            pltpu.SemaphoreType.DMA,
            pltpu.SemaphoreType.DMA,
        ],
    )
    def gk(tab_hbm, idx_hbm, out_hbm, idx_v, buf0, buf1, g0, g1):
        wid = lax.axis_index("s") * 2 + lax.axis_index("c")
        base = wid * npw
        nv = jnp.clip(Rc - base, 0, npw)
        pltpu.sync_copy(idx_hbm.at[wid], idx_v)

        @pl.when(nv > 0)
        def _():
            pltpu.async_copy(tab_hbm.at[idx_v.at[0, 0]], buf0, g0)

        def body(p, carry):
            i = 2 * p

            @pl.when(i + 1 < nv)
            def _():
                pltpu.async_copy(tab_hbm.at[idx_v.at[i + 1, 0]], buf1, g1)

            @pl.when(i < nv)
            def _():
                pltpu.make_async_copy(tab_hbm.at[idx_v.at[i, 0]],
                                      buf0, g0).wait()
                pltpu.sync_copy(buf0, out_hbm.at[pl.ds((base + i) * gr, gr)])

            @pl.when(i + 2 < nv)
            def _():
                pltpu.async_copy(tab_hbm.at[idx_v.at[i + 2, 0]], buf0, g0)

            @pl.when(i + 1 < nv)
            def _():
                pltpu.make_async_copy(tab_hbm.at[idx_v.at[i + 1, 0]],
                                      buf1, g1).wait()
                pltpu.sync_copy(buf1,
                                out_hbm.at[pl.ds((base + i + 1) * gr, gr)])

            return carry

        lax.fori_loop(0, (npw + 1) // 2, body, 0)

    return gk


def _sc_gather(table, idx):
    V, D = table.shape
    gr = min(_GR, 32768 // D)
    R = idx.shape[0]
    Rc = R // gr
    npw = -(-Rc // _NW)
    idxp = jnp.pad(idx, (0, _NW * npw * gr - R)).reshape(_NW, npw, 1, gr)
    return _sc_gather_call(V, D, Rc, gr)(table, idxp)


def _kmaj_idx(idx, n_table):
    B, N, K = idx.shape
    off = (jnp.arange(B, dtype=jnp.int32) * n_table)[:, None, None]
    return jnp.transpose(idx.astype(jnp.int32) + off, (2, 0, 1)).reshape(-1)


def _dot(a, b):
    return jnp.dot(a, b, precision=_PREC, preferred_element_type=jnp.float32)


def _padr(w, rows):
    return jnp.pad(w, ((0, rows - w.shape[0]), (0, 0)))


def _padc(w, cols):
    return jnp.pad(w, ((0, 0), (0, cols - w.shape[1])))


def _full(shape):
    return pl.BlockSpec(shape, lambda b, n: (0,) * len(shape))


def _row_spec(P, C, nb):
    return pl.BlockSpec((P, C), lambda b, n: (b * nb + n, 0))


def _gac_pre(x, v, Wgs, bgs, Wap, Wah, P, NB):
    pooled = x.ndim == 3
    BN = x.shape[1] if pooled else x.shape[0]
    C = Wah.shape[1]
    nw = len(Wgs)
    grid = (BN // (P * NB), NB)

    def body(*refs):
        it = iter(refs)
        x_ref, v_ref = next(it), next(it)
        wg = [next(it) for _ in range(nw)]
        bg = [next(it) for _ in range(nw)]
        wap, wah = next(it), next(it)
        t_ref = next(it)
        h = jnp.max(x_ref[...], axis=0) if pooled else x_ref[...]
        for W, b in zip(wg, bg):
            h = jnp.maximum(_dot(h, W[...]) + b[...], 0.0)
        q = _dot(v_ref[...], wap[...]) + _dot(h, wah[...])
        t_ref[...] = jnp.concatenate([h, q], axis=-1)

    if pooled:
        x_spec = pl.BlockSpec((x.shape[0], P, x.shape[2]),
                              lambda b, n: (0, b * NB + n, 0))
    else:
        x_spec = _row_spec(P, x.shape[1], NB)
    in_specs = [x_spec, _row_spec(P, 3, NB)]
    in_specs += [_full(W.shape) for W in Wgs]
    in_specs += [_full(b.shape) for b in bgs]
    in_specs += [_full(Wap.shape), _full(Wah.shape)]
    return pl.pallas_call(
        body, grid=grid,
        in_specs=in_specs,
        out_specs=_row_spec(P, 2 * C, NB),
        out_shape=jax.ShapeDtypeStruct((BN, 2 * C), jnp.float32),
    )(x, v, *Wgs, *bgs, Wap, Wah)


def _gac_attn(G, T, Wo, bo, ba, P, NB):
    K, BN, C2 = G.shape
    C = C2 // 2
    Cout = Wo.shape[1]
    grid = (BN // (P * NB), NB)
    aligned = C % 128 == 0

    def body(g_ref, t_ref, wo_ref, bo_ref, ba_ref, out_ref):
        g = g_ref[...]
        if aligned:
            hj, qj = g[..., :C], g[..., C:]
            e = qj - t_ref[...][None, :, C:] + ba_ref[...][None]
        else:
            hj = g
            e = g - t_ref[...][None, :, :] + ba_ref[...][None]
        e = jnp.where(e >= 0, e, 0.2 * e)
        m = jnp.max(e, axis=0, keepdims=True)
        a = jnp.exp(e - m)
        an = a / jnp.sum(a, axis=0, keepdims=True)
        if not aligned:
            an = pltpu.roll(an, C, 2)
        agg = jnp.sum(an * hj, axis=0)
        out_ref[...] = jnp.maximum(_dot(agg, wo_ref[...]) + bo_ref[...], 0.0)

    in_specs = [pl.BlockSpec((K, P, C2), lambda b, n: (0, b * NB + n, 0)),
                _row_spec(P, C2, NB),
                _full(Wo.shape), _full(bo.shape), _full(ba.shape)]
    return pl.pallas_call(
        body, grid=grid, in_specs=in_specs,
        out_specs=_row_spec(P, Cout, NB),
        out_shape=jax.ShapeDtypeStruct((BN, Cout), jnp.float32),
    )(G, T, Wo, bo, ba)


def _upsample(vf, vcT, ff, fc, W0a, W0b, b0, W1, b1, P):
    BNf = vf.shape[0]
    C1 = ff.shape[1]
    Nc, C2 = fc.shape[1], fc.shape[2]
    H2 = W1.shape[1]
    B = fc.shape[0]
    NB = BNf // (B * P)
    grid = (B, NB)

    def body(vf_ref, vcT_ref, ff_ref, fc_ref, w0a, w0b, b0r, w1, b1r,
             out_ref):
        vfb = vf_ref[...]
        vct = vcT_ref[...]
        d = jnp.zeros((P, Nc), jnp.float32)
        for mdim in range(3):
            diff = vfb[:, mdim:mdim + 1] - vct[mdim:mdim + 1, :]
            d = d + diff * diff
        iota = lax.broadcasted_iota(jnp.int32, (P, Nc), 1)
        sels, ws = [], []
        dcur = d
        for _ in range(3):
            mval = jnp.min(dcur, axis=1, keepdims=True)
            idx = jnp.min(jnp.where(dcur == mval, iota, Nc), axis=1,
                          keepdims=True)
            sel = iota == idx
            sels.append(sel)
            ws.append(1.0 / (mval + 1e-8))
            dcur = jnp.where(sel, jnp.inf, dcur)
        tot = ws[0] + ws[1] + ws[2]
        wmat = jnp.zeros((P, Nc), jnp.float32)
        for sel, w in zip(sels, ws):
            wmat = wmat + jnp.where(sel, w / tot, 0.0)
        interp = _dot(wmat, fc_ref[...])
        xx = jnp.maximum(_dot(ff_ref[...], w0a[...]) +
                         _dot(interp, w0b[...]) + b0r[...], 0.0)
        out_ref[...] = jnp.maximum(_dot(xx, w1[...]) + b1r[...], 0.0)

    in_specs = [_row_spec(P, 3, NB),
                pl.BlockSpec((None, 3, Nc), lambda b, n: (b, 0, 0)),
                _row_spec(P, C1, NB),
                pl.BlockSpec((None, Nc, C2), lambda b, n: (b, 0, 0)),
                _full(W0a.shape), _full(W0b.shape), _full(b0.shape),
                _full(W1.shape), _full(b1.shape)]
    return pl.pallas_call(
        body, grid=grid, in_specs=in_specs,
        out_specs=_row_spec(P, H2, NB),
        out_shape=jax.ShapeDtypeStruct((BNf, H2), jnp.float32),
    )(vf, vcT, ff, fc, W0a, W0b, b0, W1, b1)


def _head(f, inif, W1, b1, gamma, beta, W2p, Ssel, b2p, P, NB):
    BN, C = f.shape
    grid = (BN // (P * NB), NB)

    def body(f_ref, i_ref, w1, b1r, g, bt, w2, ssel, b2r, out_ref):
        x = _dot(f_ref[...], w1[...]) + b1r[...]
        x = jnp.maximum(g[...] * x + bt[...], 0.0)
        out_ref[...] = (_dot(x, w2[...]) + _dot(i_ref[...], ssel[...])
                        + b2r[...])

    in_specs = [_row_spec(P, C, NB), _row_spec(P, 6, NB),
                _full(W1.shape), _full(b1.shape), _full(gamma.shape),
                _full(beta.shape), _full(W2p.shape), _full(Ssel.shape),
                _full(b2p.shape)]
    return pl.pallas_call(
        body, grid=grid, in_specs=in_specs,
        out_specs=_row_spec(P, 128, NB),
        out_shape=jax.ShapeDtypeStruct((BN, 128), jnp.float32),
    )(f, inif, W1, b1, gamma, beta, W2p, Ssel, b2p)


def _final(G2, T2, Wr128, NC, P, NB):
    K, BN, _ = G2.shape
    grid = (BN // (P * NB), NB)

    def body(g2_ref, t2_ref, wr_ref, out_ref):
        g2 = g2_ref[...]
        dij = g2 - t2_ref[...][None, :, :]
        logits = _dot(dij.reshape(K * P, 128),
                      wr_ref[...]).reshape(K, P, 128)
        e = jnp.where(logits >= 0, logits, 0.2 * logits)
        m = jnp.max(e, axis=0, keepdims=True)
        a = jnp.exp(e - m)
        z = jnp.sum(a, axis=0)
        s = jnp.sum(a * g2, axis=0) / z
        mask = lax.broadcasted_iota(jnp.int32, (P, 128), 1) < NC
        zz = jnp.where(mask, s, -jnp.inf)
        mm = jnp.max(zz, axis=1, keepdims=True)
        lse = mm + jnp.log(jnp.sum(jnp.exp(zz - mm), axis=1, keepdims=True))
        out_ref[...] = s - lse

    in_specs = [pl.BlockSpec((K, P, 128), lambda b, n: (0, b * NB + n, 0)),
                _row_spec(P, 128, NB), _full(Wr128.shape)]
    return pl.pallas_call(
        body, grid=grid, in_specs=in_specs,
        out_specs=_row_spec(P, 128, NB),
        out_shape=jax.ShapeDtypeStruct((BN, 128), jnp.float32),
    )(G2, T2, Wr128)


_P_PRE = [1024, 512, 512, 128, 64]
_P_ATTN = [1024, 256, 128, 32, 64]
_P_UP = [512, 256, 128, 128]


def kernel(features, vertex0, vertex1, vertex2, vertex3, vertex4,
           adjids0, adjids1, adjids2, adjids3, adjids4,
           cmap0, cmap1, cmap2, cmap3, params):
    args = (features, vertex0, vertex1, vertex2, vertex3, vertex4,
            adjids0, adjids1, adjids2, adjids3, adjids4,
            cmap0, cmap1, cmap2, cmap3)
    outs = [_forward(*(a[b:b + 1] for a in args), params)
            for b in range(features.shape[0])]
    return jnp.concatenate(outs, axis=0)


def _forward(features, vertex0, vertex1, vertex2, vertex3, vertex4,
             adjids0, adjids1, adjids2, adjids3, adjids4,
             cmap0, cmap1, cmap2, cmap3, params):
    vs = [vertex0, vertex1, vertex2, vertex3, vertex4]
    adjs = [adjids0, adjids1, adjids2, adjids3, adjids4]
    cmaps = [cmap0, cmap1, cmap2, cmap3]
    B = features.shape[0]
    ns = [v.shape[1] for v in vs]
    vflat = [v.reshape(B * v.shape[1], 3) for v in vs]

    inif = features[:, :, 0:6].reshape(B * ns[0], 6)
    x = features[:, :, 2:6].reshape(B * ns[0], 4)
    prd = []
    fo = None
    for l in range(5):
        gp = params['gac%d' % l]
        C = gp['Wa'].shape[1]
        aligned = C % 128 == 0
        Wgs = list(gp['Wg'])
        bgs = [b.reshape(1, -1) for b in gp['bg']]
        if Wgs[0].shape[0] != x.shape[-1]:
            Wgs[0] = _padr(Wgs[0], x.shape[-1])
        Wap, Wah = gp['Wa'][:3], gp['Wa'][3:]
        Cout = gp['Wo'].shape[1]
        Cot = max(Cout, 128)
        if aligned:
            ba = gp['ba'].reshape(1, -1)
            Wo = gp['Wo']
        else:
            ba = jnp.pad(gp['ba'], (C, 0)).reshape(1, -1)
            Wo = _padr(gp['Wo'], 2 * C)
        Wo = _padc(Wo, Cot)
        bo = _padc(gp['bo'].reshape(1, -1), Cot)
        nbl = ns[l] // _P_PRE[l]
        T = _gac_pre(x, vflat[l], Wgs, bgs, Wap, Wah, _P_PRE[l], nbl)
        K = adjs[l].shape[2]
        G = _sc_gather(T, _kmaj_idx(adjs[l], ns[l]))
        fo = _gac_attn(G.reshape(K, B * ns[l], 2 * C), T, Wo, bo, ba,
                       _P_ATTN[l], ns[l] // _P_ATTN[l])
        if l < 4:
            prd.append(fo)
            S = cmaps[l].shape[2]
            Gp = _sc_gather(fo, _kmaj_idx(cmaps[l], ns[l]))
            x = Gp.reshape(S, B * ns[l + 1], Cot)

    fcur = fo
    for l in [3, 2, 1, 0]:
        up = params['up%d' % l]
        C2 = fcur.shape[1]
        C1 = up['W'][0].shape[0] - C2
        W0a, W0b = up['W'][0][:C1], up['W'][0][C1:]
        if W0a.shape[0] != prd[l].shape[1]:
            W0a = _padr(W0a, prd[l].shape[1])
        fcur = _upsample(vflat[l],
                         jnp.swapaxes(vs[l + 1], 1, 2), prd[l],
                         fcur.reshape(B, ns[l + 1], C2),
                         W0a, W0b, up['b'][0].reshape(1, -1),
                         up['W'][1], up['b'][1].reshape(1, -1), _P_UP[l])

    NC = params['W2'].shape[1]
    W2p = _padc(params['W2'], 128)
    b2p = _padc(params['b2'].reshape(1, -1), 128)
    Ssel = jnp.pad(jnp.eye(6, dtype=jnp.float32), ((0, 0), (16, 106)))
    Wr128 = jnp.pad(params['Wr'], ((16, 106), (0, 128 - NC)))
    T2 = _head(fcur, inif, params['W1'], params['b1'].reshape(1, -1),
               params['gamma'].reshape(1, -1), params['beta'].reshape(1, -1),
               W2p, Ssel, b2p, 1024, ns[0] // 1024)
    K0 = adjs[0].shape[2]
    G2 = _sc_gather(T2, _kmaj_idx(adjs[0], ns[0]))
    out = _final(G2.reshape(K0, B * ns[0], 128), T2, Wr128, NC,
                 512, ns[0] // 512)
    return out.reshape(B, ns[0], 128)[:, :, :NC]
